# baseline (speedup 1.0000x reference)
import jax
import jax.numpy as jnp
from jax.experimental import pallas as pl
from jax.experimental.pallas import tpu as pltpu

_REG_MAX = 16
_TOPK = 10
_BOX_W = 7.5
_OBJ_W = 1.0
_NEG = float("-inf")
_SUB = 8

_ATAN_COEF = (
    0.9999999999902919, -0.33333332995051296, 0.19999980353689645,
    -0.14285262492495704, 0.11105656189675474, -0.09051137251409,
    0.07502231366742305, -0.06038548449194854, 0.04390286868997824,
    -0.026271574631780946, 0.011602323441057973, -0.003261486111460649,
    0.00043016480682746657,
)
_HALF_PI = 1.5707963267948966


def _atan_pos(z):
    inv = z > 1.0
    x = jnp.where(inv, 1.0 / jnp.maximum(z, 1e-30), z)
    t = x * x
    acc = jnp.full_like(t, _ATAN_COEF[-1])
    for c in _ATAN_COEF[-2::-1]:
        acc = acc * t + c
    r = x * acc
    return jnp.where(inv, _HALF_PI - r, r)


def _loss_kernel(boxes_ref, scores_ref, targets_ref, box_out_ref, obj_out_ref):
    x = boxes_ref[0]
    sub, lanes = x.shape[1], x.shape[2]
    n = sub * lanes

    ds = []
    for k in range(4):
        xe = jnp.exp(x[k * _REG_MAX:(k + 1) * _REG_MAX])
        den = jnp.sum(xe, axis=0)
        w = jax.lax.broadcasted_iota(
            jnp.int32, (_REG_MAX, sub, lanes), 0).astype(jnp.float32)
        num = jnp.sum(xe * w, axis=0)
        ds.append(num / den)
    l, t, r, b = ds
    b1x1 = -l
    b1y1 = -t
    b1x2 = r
    b1y2 = b

    eps = 1e-7
    w1 = b1x2 - b1x1
    h1 = b1y2 - b1y1
    w1h1 = w1 * h1
    sx1 = b1x1 + b1x2
    sy1 = b1y1 + b1y2
    a1 = _atan_pos(w1 / (h1 + eps))

    tgt = targets_ref[0]
    m_gt = tgt.shape[0]
    b2x1 = tgt[:, 0].reshape(m_gt, 1, 1)
    b2y1 = tgt[:, 1].reshape(m_gt, 1, 1)
    b2x2 = tgt[:, 2].reshape(m_gt, 1, 1)
    b2y2 = tgt[:, 3].reshape(m_gt, 1, 1)
    w2 = b2x2 - b2x1
    h2 = b2y2 - b2y1
    w2h2e = w2 * h2 + eps
    sx2 = b2x1 + b2x2
    sy2 = b2y1 + b2y2
    a2 = _atan_pos(w2 / (h2 + eps))

    b1x1e = b1x1[None]
    b1y1e = b1y1[None]
    b1x2e = b1x2[None]
    b1y2e = b1y2[None]
    inter_w = jnp.maximum(jnp.minimum(b1x2e, b2x2) - jnp.maximum(b1x1e, b2x1), 0.0)
    inter_h = jnp.maximum(jnp.minimum(b1y2e, b2y2) - jnp.maximum(b1y1e, b2y1), 0.0)
    inter = inter_w * inter_h
    union = (w1h1[None] + w2h2e) - inter
    iou = inter / union
    cw = jnp.maximum(b1x2e, b2x2) - jnp.minimum(b1x1e, b2x1)
    ch = jnp.maximum(b1y2e, b2y2) - jnp.minimum(b1y1e, b2y1)
    c2 = cw * cw + ch * ch + eps
    dx = sx2 - sx1[None]
    dy = sy2 - sy1[None]
    rho2 = (dx * dx + dy * dy) * 0.25
    da = a2 - a1[None]
    v = (4.0 / (jnp.pi ** 2)) * (da * da)
    alpha_den = v - iou + (1.0 + eps)
    ciou = iou - (rho2 / c2 + (v * v) / alpha_den)

    work = ciou
    thr = None
    for i in range(_TOPK):
        thr = jnp.max(jnp.max(work, axis=2, keepdims=True), axis=1, keepdims=True)
        if i < _TOPK - 1:
            work = jnp.where(work == thr, _NEG, work)

    selv = jnp.where(ciou >= thr, ciou, _NEG)
    m = jnp.max(selv, axis=0)
    posf = (m > 0.0).astype(jnp.float32)
    it = jnp.maximum(m, 0.0)

    npos = jnp.sum(posf)
    sbox = jnp.sum((1.0 - it) * posf)
    box_b = jnp.where(npos > 0.0, sbox / jnp.maximum(npos, 1.0), 0.0)

    s = scores_ref[0]
    softplus = jnp.log1p(jnp.exp(-jnp.abs(s))) + jnp.maximum(s, 0.0)
    obj_b = (jnp.sum(softplus) - jnp.sum(s * it)) / n

    box_out_ref[0] = jnp.reshape(box_b, (1, 1))
    obj_out_ref[0] = jnp.reshape(obj_b, (1, 1))


@jax.jit
def kernel(boxes, scores, targets):
    bsz, c, n = boxes.shape
    m = targets.shape[1]
    lanes = n // _SUB
    boxes_r = boxes.reshape(bsz, c, _SUB, lanes)
    scores_r = scores.reshape(bsz, _SUB, lanes)
    box_b, obj_b = pl.pallas_call(
        _loss_kernel,
        grid=(bsz,),
        in_specs=[
            pl.BlockSpec((1, c, _SUB, lanes), lambda i: (i, 0, 0, 0)),
            pl.BlockSpec((1, _SUB, lanes), lambda i: (i, 0, 0)),
            pl.BlockSpec((1, m, 4), lambda i: (i, 0, 0)),
        ],
        out_specs=[
            pl.BlockSpec((1, 1, 1), lambda i: (i, 0, 0)),
            pl.BlockSpec((1, 1, 1), lambda i: (i, 0, 0)),
        ],
        out_shape=[
            jax.ShapeDtypeStruct((bsz, 1, 1), jnp.float32),
            jax.ShapeDtypeStruct((bsz, 1, 1), jnp.float32),
        ],
        compiler_params=pltpu.CompilerParams(
            dimension_semantics=("arbitrary",),
        ),
    )(boxes_r, scores_r, targets)
    tb = jnp.sum(box_b)
    to = jnp.sum(obj_b)
    total = (_BOX_W * tb + _OBJ_W * to) / bsz
    return total, jax.lax.stop_gradient(tb), jax.lax.stop_gradient(to)


# speedup vs baseline: 1.5641x; 1.5641x over previous
import jax
import jax.numpy as jnp
from jax.experimental import pallas as pl
from jax.experimental.pallas import tpu as pltpu

_REG_MAX = 16
_TOPK = 10
_BOX_W = 7.5
_OBJ_W = 1.0
_NEG = float("-inf")
_SUB = 8

_ATAN_COEF = (
    0.9999999999902919, -0.33333332995051296, 0.19999980353689645,
    -0.14285262492495704, 0.11105656189675474, -0.09051137251409,
    0.07502231366742305, -0.06038548449194854, 0.04390286868997824,
    -0.026271574631780946, 0.011602323441057973, -0.003261486111460649,
    0.00043016480682746657,
)
_HALF_PI = 1.5707963267948966


def _atan_pos(z):
    inv = z > 1.0
    x = jnp.where(inv, 1.0 / jnp.maximum(z, 1e-30), z)
    t = x * x
    acc = jnp.full_like(t, _ATAN_COEF[-1])
    for c in _ATAN_COEF[-2::-1]:
        acc = acc * t + c
    r = x * acc
    return jnp.where(inv, _HALF_PI - r, r)


def _loss_kernel(boxes_ref, scores_ref, targets_ref, box_out_ref, obj_out_ref):
    x2 = boxes_ref[0]
    n = x2.shape[1]
    sub = _SUB
    lanes = n // _SUB
    x = x2.reshape(x2.shape[0], sub, lanes)

    ds = []
    for k in range(4):
        xe = jnp.exp(x[k * _REG_MAX:(k + 1) * _REG_MAX])
        den = jnp.sum(xe, axis=0)
        w = jax.lax.broadcasted_iota(
            jnp.int32, (_REG_MAX, sub, lanes), 0).astype(jnp.float32)
        num = jnp.sum(xe * w, axis=0)
        ds.append(num / den)
    l, t, r, b = ds
    b1x1 = -l
    b1y1 = -t
    b1x2 = r
    b1y2 = b

    eps = 1e-7
    w1 = b1x2 - b1x1
    h1 = b1y2 - b1y1
    w1h1 = w1 * h1
    sx1 = b1x1 + b1x2
    sy1 = b1y1 + b1y2
    a1 = _atan_pos(w1 / (h1 + eps))

    tgt = targets_ref[0]
    m_gt = tgt.shape[0]
    b2x1 = tgt[:, 0].reshape(m_gt, 1, 1)
    b2y1 = tgt[:, 1].reshape(m_gt, 1, 1)
    b2x2 = tgt[:, 2].reshape(m_gt, 1, 1)
    b2y2 = tgt[:, 3].reshape(m_gt, 1, 1)
    w2 = b2x2 - b2x1
    h2 = b2y2 - b2y1
    w2h2e = w2 * h2 + eps
    sx2 = b2x1 + b2x2
    sy2 = b2y1 + b2y2
    a2 = _atan_pos(w2 / (h2 + eps))

    b1x1e = b1x1[None]
    b1y1e = b1y1[None]
    b1x2e = b1x2[None]
    b1y2e = b1y2[None]
    inter_w = jnp.maximum(jnp.minimum(b1x2e, b2x2) - jnp.maximum(b1x1e, b2x1), 0.0)
    inter_h = jnp.maximum(jnp.minimum(b1y2e, b2y2) - jnp.maximum(b1y1e, b2y1), 0.0)
    inter = inter_w * inter_h
    union = (w1h1[None] + w2h2e) - inter
    iou = inter / union
    cw = jnp.maximum(b1x2e, b2x2) - jnp.minimum(b1x1e, b2x1)
    ch = jnp.maximum(b1y2e, b2y2) - jnp.minimum(b1y1e, b2y1)
    c2 = cw * cw + ch * ch + eps
    dx = sx2 - sx1[None]
    dy = sy2 - sy1[None]
    rho2 = (dx * dx + dy * dy) * 0.25
    da = a2 - a1[None]
    v = (4.0 / (jnp.pi ** 2)) * (da * da)
    alpha_den = v - iou + (1.0 + eps)
    ciou = iou - (rho2 / c2 + (v * v) / alpha_den)

    work = ciou
    thr = None
    for i in range(_TOPK):
        thr = jnp.max(jnp.max(work, axis=2, keepdims=True), axis=1, keepdims=True)
        if i < _TOPK - 1:
            work = jnp.where(work == thr, _NEG, work)

    selv = jnp.where(ciou >= thr, ciou, _NEG)
    m = jnp.max(selv, axis=0)
    posf = (m > 0.0).astype(jnp.float32)
    it = jnp.maximum(m, 0.0)

    npos = jnp.sum(posf)
    sbox = jnp.sum((1.0 - it) * posf)
    box_b = jnp.where(npos > 0.0, sbox / jnp.maximum(npos, 1.0), 0.0)

    s = scores_ref[0]
    softplus = jnp.log1p(jnp.exp(-jnp.abs(s))) + jnp.maximum(s, 0.0)
    obj_b = (jnp.sum(softplus) - jnp.sum(s * it)) / n

    box_out_ref[0] = jnp.reshape(box_b, (1, 1))
    obj_out_ref[0] = jnp.reshape(obj_b, (1, 1))


@jax.jit
def kernel(boxes, scores, targets):
    bsz, c, n = boxes.shape
    m = targets.shape[1]
    box_b, obj_b = pl.pallas_call(
        _loss_kernel,
        grid=(bsz,),
        in_specs=[
            pl.BlockSpec((1, c, n), lambda i: (i, 0, 0)),
            pl.BlockSpec((1, _SUB, n // _SUB), lambda i: (i, 0, 0)),
            pl.BlockSpec((1, m, 4), lambda i: (i, 0, 0)),
        ],
        out_specs=[
            pl.BlockSpec((1, 1, 1), lambda i: (i, 0, 0)),
            pl.BlockSpec((1, 1, 1), lambda i: (i, 0, 0)),
        ],
        out_shape=[
            jax.ShapeDtypeStruct((bsz, 1, 1), jnp.float32),
            jax.ShapeDtypeStruct((bsz, 1, 1), jnp.float32),
        ],
        compiler_params=pltpu.CompilerParams(
            dimension_semantics=("arbitrary",),
        ),
    )(boxes, scores.reshape(bsz, _SUB, n // _SUB), targets)
    tb = jnp.sum(box_b)
    to = jnp.sum(obj_b)
    total = (_BOX_W * tb + _OBJ_W * to) / bsz
    return total, jax.lax.stop_gradient(tb), jax.lax.stop_gradient(to)


# speedup vs baseline: 1.8933x; 1.2105x over previous
import jax
import jax.numpy as jnp
from jax.experimental import pallas as pl
from jax.experimental.pallas import tpu as pltpu

_REG_MAX = 16
_TOPK = 10
_BOX_W = 7.5
_OBJ_W = 1.0
_NEG = float("-inf")
_SUB = 8

_ATAN_COEF = (
    0.9999999999902919, -0.33333332995051296, 0.19999980353689645,
    -0.14285262492495704, 0.11105656189675474, -0.09051137251409,
    0.07502231366742305, -0.06038548449194854, 0.04390286868997824,
    -0.026271574631780946, 0.011602323441057973, -0.003261486111460649,
    0.00043016480682746657,
)
_HALF_PI = 1.5707963267948966


def _atan_pos(z):
    inv = z > 1.0
    x = jnp.where(inv, 1.0 / jnp.maximum(z, 1e-30), z)
    t = x * x
    acc = jnp.full_like(t, _ATAN_COEF[-1])
    for c in _ATAN_COEF[-2::-1]:
        acc = acc * t + c
    r = x * acc
    return jnp.where(inv, _HALF_PI - r, r)


def _make_loss_kernel(bsz):
    def _loss_kernel(boxes_ref, scores_ref, targets_ref,
                     tot_ref, box_ref, obj_ref):
        x2 = boxes_ref[0]
        n = x2.shape[1]
        lanes = n // _SUB

        rr = jax.lax.broadcasted_iota(jnp.int32, (2 * 4, 4 * _REG_MAX), 0)
        cc = jax.lax.broadcasted_iota(jnp.int32, (2 * 4, 4 * _REG_MAX), 1)
        side = jnp.where(rr < 4, rr, rr - 4)
        match = (cc // _REG_MAX) == side
        proj = jnp.where(rr < 4, 1, cc % _REG_MAX).astype(jnp.float32)
        w_mat = jnp.where(match, proj, 0.0)

        xe = jnp.exp(x2)
        xe_hi = xe.astype(jnp.bfloat16)
        xe_lo = (xe - xe_hi.astype(jnp.float32)).astype(jnp.bfloat16)
        w_bf = w_mat.astype(jnp.bfloat16)
        dn = (((1,), (0,)), ((), ()))
        p = (jax.lax.dot_general(w_bf, xe_hi, dn,
                                 preferred_element_type=jnp.float32)
             + jax.lax.dot_general(w_bf, xe_lo, dn,
                                   preferred_element_type=jnp.float32))
        d = p[4:8] / p[0:4]
        d3 = d.reshape(4, _SUB, lanes)
        l = d3[0]
        t = d3[1]
        r = d3[2]
        b = d3[3]
        b1x1 = -l
        b1y1 = -t
        b1x2 = r
        b1y2 = b

        eps = 1e-7
        w1 = b1x2 - b1x1
        h1 = b1y2 - b1y1
        w1h1 = w1 * h1
        sx1 = b1x1 + b1x2
        sy1 = b1y1 + b1y2
        a1 = _atan_pos(w1 / (h1 + eps))

        tgt = targets_ref[0]
        m_gt = tgt.shape[0]
        b2x1 = tgt[:, 0].reshape(m_gt, 1, 1)
        b2y1 = tgt[:, 1].reshape(m_gt, 1, 1)
        b2x2 = tgt[:, 2].reshape(m_gt, 1, 1)
        b2y2 = tgt[:, 3].reshape(m_gt, 1, 1)
        w2 = b2x2 - b2x1
        h2 = b2y2 - b2y1
        w2h2e = w2 * h2 + eps
        sx2 = b2x1 + b2x2
        sy2 = b2y1 + b2y2
        a2 = _atan_pos(w2 / (h2 + eps))

        b1x1e = b1x1[None]
        b1y1e = b1y1[None]
        b1x2e = b1x2[None]
        b1y2e = b1y2[None]
        inter_w = jnp.maximum(
            jnp.minimum(b1x2e, b2x2) - jnp.maximum(b1x1e, b2x1), 0.0)
        inter_h = jnp.maximum(
            jnp.minimum(b1y2e, b2y2) - jnp.maximum(b1y1e, b2y1), 0.0)
        inter = inter_w * inter_h
        union = (w1h1[None] + w2h2e) - inter
        iou = inter / union
        cw = jnp.maximum(b1x2e, b2x2) - jnp.minimum(b1x1e, b2x1)
        ch = jnp.maximum(b1y2e, b2y2) - jnp.minimum(b1y1e, b2y1)
        c2 = cw * cw + ch * ch + eps
        dx = sx2 - sx1[None]
        dy = sy2 - sy1[None]
        rho2 = (dx * dx + dy * dy) * 0.25
        da = a2 - a1[None]
        v = (4.0 / (jnp.pi ** 2)) * (da * da)
        alpha_den = v - iou + (1.0 + eps)
        pen_num = rho2 * alpha_den + (v * v) * c2
        ciou = iou - pen_num / (c2 * alpha_den)

        work = ciou
        thr = None
        for i in range(_TOPK):
            thr = jnp.max(jnp.max(work, axis=2, keepdims=True),
                          axis=1, keepdims=True)
            if i < _TOPK - 1:
                work = jnp.where(work == thr, _NEG, work)

        selv = jnp.where(ciou >= thr, ciou, _NEG)
        m = jnp.max(selv, axis=0)
        posf = (m > 0.0).astype(jnp.float32)
        it = jnp.maximum(m, 0.0)

        npos = jnp.sum(posf)
        sum_it = jnp.sum(it)
        box_b = jnp.where(npos > 0.0,
                          (npos - sum_it) / jnp.maximum(npos, 1.0), 0.0)

        s = scores_ref[0]
        softplus = jnp.log1p(jnp.exp(-jnp.abs(s))) + jnp.maximum(s, 0.0)
        obj_b = (jnp.sum(softplus) - jnp.sum(s * it)) / n

        bi = pl.program_id(0)

        @pl.when(bi == 0)
        def _init():
            box_ref[0] = jnp.zeros((1, 1), jnp.float32)
            obj_ref[0] = jnp.zeros((1, 1), jnp.float32)

        box_ref[0] += jnp.reshape(box_b, (1, 1))
        obj_ref[0] += jnp.reshape(obj_b, (1, 1))

        @pl.when(bi == bsz - 1)
        def _fin():
            tot_ref[0] = (_BOX_W * box_ref[0] + _OBJ_W * obj_ref[0]) / bsz

    return _loss_kernel


@jax.jit
def kernel(boxes, scores, targets):
    bsz, c, n = boxes.shape
    m = targets.shape[1]
    tot, tb, to = pl.pallas_call(
        _make_loss_kernel(bsz),
        grid=(bsz,),
        in_specs=[
            pl.BlockSpec((1, c, n), lambda i: (i, 0, 0)),
            pl.BlockSpec((1, _SUB, n // _SUB), lambda i: (i, 0, 0)),
            pl.BlockSpec((1, m, 4), lambda i: (i, 0, 0)),
        ],
        out_specs=[
            pl.BlockSpec((1, 1, 1), lambda i: (0, 0, 0)),
            pl.BlockSpec((1, 1, 1), lambda i: (0, 0, 0)),
            pl.BlockSpec((1, 1, 1), lambda i: (0, 0, 0)),
        ],
        out_shape=[
            jax.ShapeDtypeStruct((1, 1, 1), jnp.float32),
            jax.ShapeDtypeStruct((1, 1, 1), jnp.float32),
            jax.ShapeDtypeStruct((1, 1, 1), jnp.float32),
        ],
        compiler_params=pltpu.CompilerParams(
            dimension_semantics=("arbitrary",),
        ),
    )(boxes, scores.reshape(bsz, _SUB, n // _SUB), targets)
    total = tot[0, 0, 0]
    tb_s = tb[0, 0, 0]
    to_s = to[0, 0, 0]
    return total, jax.lax.stop_gradient(tb_s), jax.lax.stop_gradient(to_s)
